# Initial kernel scaffold; baseline (speedup 1.0000x reference)
import jax
import jax.numpy as jnp
from jax.experimental import pallas as pl


def kernel(attn_bias, spatial_pos, x, edge_input, attn_edge_type, edge_encoder_w, edge_dis_encoder_w, spatial_pos_encoder_w, graph_token_w):
    raise NotImplementedError("write your pallas kernel here")


# speedup vs baseline: 18.2189x; 18.2189x over previous
import functools

import jax
import jax.numpy as jnp
from jax import lax
from jax.experimental import pallas as pl
from jax.experimental.pallas import tpu as pltpu
from jax.experimental.pallas import tpu_sc as plsc

H = 32
D_MAX = 5
F = 3
E_ROWS = 1537
E_STRIDE = 1544
SP_ROWS = 512
SP_BASE = 25 * E_STRIDE
T_ROWS = SP_BASE + SP_ROWS

NC, NS = 2, 16
NW = NC * NS
C_PAIRS = 64
ROWS_PER_CHUNK = C_PAIRS * 16


def _table_body(ew_ref, dis_ref, spw_ref, out_ref):
    ew = ew_ref[...]
    for s in range(5):
        for d in range(5):
            w = jnp.dot(ew, dis_ref[d], preferred_element_type=jnp.float32)
            w = w * (1.0 / (3.0 * (s + 1)))
            out_ref[pl.ds((s * 5 + d) * E_STRIDE, E_ROWS), :] = w
    out_ref[pl.ds(SP_BASE, SP_ROWS), :] = spw_ref[...]


def _build_table(ew, dis5, spw):
    return pl.pallas_call(
        _table_body,
        out_shape=jax.ShapeDtypeStruct((T_ROWS, H), jnp.float32),
    )(ew, dis5, spw)


def _idx_body(sp_ref, edge_ref, out_ref):
    sp = sp_ref[...][:, 0]
    spc = jnp.clip(sp - 1, 1, 5)
    base = (spc - 1) * (5 * E_STRIDE)
    d_off = (jnp.arange(D_MAX * F, dtype=jnp.int32) // F) * E_STRIDE
    out_ref[:, 0:D_MAX * F] = edge_ref[...] + d_off[None, :] + base[:, None]
    out_ref[:, D_MAX * F:D_MAX * F + 1] = sp_ref[...] + SP_BASE


def _build_idx(sp_r, edge_r, p_total):
    blk = 1024
    grid = p_total // blk
    return pl.pallas_call(
        _idx_body,
        grid=(grid,),
        in_specs=[
            pl.BlockSpec((blk, 1), lambda i: (i, 0)),
            pl.BlockSpec((blk, D_MAX * F), lambda i: (i, 0)),
        ],
        out_specs=pl.BlockSpec((blk, D_MAX * F + 1), lambda i: (i, 0)),
        out_shape=jax.ShapeDtypeStruct((p_total, D_MAX * F + 1), jnp.int32),
    )(sp_r, edge_r)


def _sc_gather_sum(table, idx2, p_total):
    ppw = p_total // NW
    nchunk = ppw // C_PAIRS
    mesh = plsc.VectorSubcoreMesh(core_axis_name="c", subcore_axis_name="s")

    @functools.partial(
        pl.kernel,
        out_type=jax.ShapeDtypeStruct((p_total, H), jnp.float32),
        mesh=mesh,
        compiler_params=pltpu.CompilerParams(use_tc_tiling_on_sc=False),
        scratch_types=[
            pltpu.VMEM((ROWS_PER_CHUNK // 128, 128), jnp.int32),
            pltpu.VMEM((ROWS_PER_CHUNK, H), jnp.float32),
            pltpu.VMEM((C_PAIRS, H), jnp.float32),
            pltpu.SemaphoreType.DMA,
        ],
    )
    def k(table_hbm, idx_hbm, out_hbm, idx_v, rows_v, out_v, sem):
        wid = lax.axis_index("s") * NC + lax.axis_index("c")

        def chunk_body(ci, carry):
            pair0 = pl.multiple_of(wid * ppw + ci * C_PAIRS, C_PAIRS)
            irow0 = pl.multiple_of(pair0 * 16 // 128, C_PAIRS * 16 // 128)
            pltpu.sync_copy(idx_hbm.at[pl.ds(irow0, ROWS_PER_CHUNK // 128)],
                            idx_v)
            cps = [
                pltpu.async_copy(table_hbm.at[idx_v.at[j]],
                                 rows_v.at[pl.ds(j * 128, 128)], sem)
                for j in range(ROWS_PER_CHUNK // 128)
            ]
            for cp in cps:
                cp.wait()

            def pair_body(p, c2):
                r0 = p * 16
                a = rows_v[r0, 0:16]
                b = rows_v[r0, 16:32]
                for k_ in range(1, 16):
                    a = a + rows_v[r0 + k_, 0:16]
                    b = b + rows_v[r0 + k_, 16:32]
                out_v[p, 0:16] = a
                out_v[p, 16:32] = b
                return c2

            lax.fori_loop(0, C_PAIRS, pair_body, 0)
            pltpu.sync_copy(out_v, out_hbm.at[pl.ds(pair0, C_PAIRS)])
            return carry

        lax.fori_loop(0, nchunk, chunk_body, 0)

    return k(table, idx2)


def _asm_body(ab_ref, int_ref, tok_ref, out_ref):
    x = int_ref[0]
    ii = lax.broadcasted_iota(jnp.int32, (H, H), 0)
    jj = lax.broadcasted_iota(jnp.int32, (H, H), 1)
    eye = (ii == jj).astype(jnp.float32)
    t = lax.dot_general(eye, x, (((1,), (1,)), ((), ())),
                        preferred_element_type=jnp.float32)
    n = ab_ref.shape[1] - 1
    t = t.reshape(H, n, n)
    ab = ab_ref[0]
    tok = tok_ref[0, :]
    interior = t + 2.0 * ab[1:, 1:][None, :, :]
    col0 = 2.0 * ab[1:, 0][None, :] + tok[:, None]
    row0 = 2.0 * ab[0, :][None, :] + tok[:, None]
    body = jnp.concatenate([col0[:, :, None], interior], axis=2)
    out = jnp.concatenate([row0[:, None, :], body], axis=1)
    out_ref[0] = out


def _assemble(attn_bias, interior3, gtw):
    b, np1, _ = attn_bias.shape
    n = np1 - 1
    return pl.pallas_call(
        _asm_body,
        grid=(b,),
        in_specs=[
            pl.BlockSpec((1, np1, np1), lambda i: (i, 0, 0)),
            pl.BlockSpec((1, n * n, H), lambda i: (i, 0, 0)),
            pl.BlockSpec((1, H), lambda i: (0, 0)),
        ],
        out_specs=pl.BlockSpec((1, H, np1, np1), lambda i: (i, 0, 0, 0)),
        out_shape=jax.ShapeDtypeStruct((b, H, np1, np1), jnp.float32),
    )(attn_bias, interior3, gtw)


def kernel(attn_bias, spatial_pos, x, edge_input, attn_edge_type,
           edge_encoder_w, edge_dis_encoder_w, spatial_pos_encoder_w,
           graph_token_w):
    b, np1, _ = attn_bias.shape
    n = np1 - 1
    p_total = b * n * n

    dis5 = edge_dis_encoder_w.reshape(-1, H, H)[:D_MAX]
    table = _build_table(edge_encoder_w, dis5, spatial_pos_encoder_w)

    sp_r = spatial_pos.astype(jnp.int32).reshape(p_total, 1)
    edge_r = edge_input.astype(jnp.int32)[..., :D_MAX, :].reshape(
        p_total, D_MAX * F)
    idx = _build_idx(sp_r, edge_r, p_total)
    idx2 = idx.reshape(p_total * 16 // 128, 128)

    interior = _sc_gather_sum(table, idx2, p_total)
    return _assemble(attn_bias, interior.reshape(b, n * n, H), graph_token_w)


# speedup vs baseline: 22.9318x; 1.2587x over previous
import functools

import jax
import jax.numpy as jnp
from jax import lax
from jax.experimental import pallas as pl
from jax.experimental.pallas import tpu as pltpu
from jax.experimental.pallas import tpu_sc as plsc

H = 32
D_MAX = 5
F = 3
E_ROWS = 1537
E_STRIDE = 1552
SP_ROWS = 512
SP_BASE = 25 * E_STRIDE
T_ROWS = SP_BASE + SP_ROWS

NC, NS = 2, 16
NW = NC * NS
C_PAIRS = 64
RPC = C_PAIRS * 16
NSTREAM = RPC // 128


def _table_body(ew_ref, dis_ref, spw_ref, out_ref):
    ew = ew_ref[...]
    for s in range(5):
        for d in range(5):
            w = jnp.dot(ew, dis_ref[d], preferred_element_type=jnp.float32)
            w = w * (1.0 / (3.0 * (s + 1)))
            out_ref[pl.ds((s * 5 + d) * E_STRIDE, E_ROWS), :] = w.astype(jnp.bfloat16)
    out_ref[pl.ds(SP_BASE, SP_ROWS), :] = spw_ref[...].astype(jnp.bfloat16)


def _build_table(ew, dis5, spw):
    return pl.pallas_call(
        _table_body,
        out_shape=jax.ShapeDtypeStruct((T_ROWS, H), jnp.bfloat16),
    )(ew, dis5, spw)


def _idx_body(sp_ref, edge_ref, out_ref):
    sp = sp_ref[...][:, 0]
    spc = jnp.clip(sp - 1, 1, 5)
    base = (spc - 1) * (5 * E_STRIDE)
    d_off = (jnp.arange(D_MAX * F, dtype=jnp.int32) // F) * E_STRIDE
    out_ref[:, 0:D_MAX * F] = edge_ref[...] + d_off[None, :] + base[:, None]
    out_ref[:, D_MAX * F:D_MAX * F + 1] = sp_ref[...] + SP_BASE


def _build_idx(sp_r, edge_r, p_total):
    blk = 1024
    grid = p_total // blk
    return pl.pallas_call(
        _idx_body,
        grid=(grid,),
        in_specs=[
            pl.BlockSpec((blk, 1), lambda i: (i, 0)),
            pl.BlockSpec((blk, D_MAX * F), lambda i: (i, 0)),
        ],
        out_specs=pl.BlockSpec((blk, D_MAX * F + 1), lambda i: (i, 0)),
        out_shape=jax.ShapeDtypeStruct((p_total, D_MAX * F + 1), jnp.int32),
    )(sp_r, edge_r)


def _sc_gather_sum(table, idx2, p_total):
    ppw = p_total // NW
    nchunk = ppw // C_PAIRS
    nhalf = nchunk // 2
    mesh = plsc.VectorSubcoreMesh(core_axis_name="c", subcore_axis_name="s")

    @functools.partial(
        pl.kernel,
        out_type=jax.ShapeDtypeStruct((p_total, H), jnp.bfloat16),
        mesh=mesh,
        compiler_params=pltpu.CompilerParams(use_tc_tiling_on_sc=False),
        scratch_types=[
            pltpu.VMEM((NSTREAM, 128), jnp.int32),
            pltpu.VMEM((NSTREAM, 128), jnp.int32),
            pltpu.VMEM((RPC, H), jnp.bfloat16),
            pltpu.VMEM((RPC, H), jnp.bfloat16),
            pltpu.VMEM((C_PAIRS, H), jnp.bfloat16),
            pltpu.VMEM((C_PAIRS, H), jnp.bfloat16),
            pltpu.SemaphoreType.DMA,
            pltpu.SemaphoreType.DMA,
        ],
    )
    def k(table_hbm, idx_hbm, out_hbm,
          idx_v0, idx_v1, rows_v0, rows_v1, out_v0, out_v1, sem0, sem1):
        wid = lax.axis_index("s") * NC + lax.axis_index("c")
        base_pair = wid * ppw

        def load_idx(ci, idx_v):
            pair0 = pl.multiple_of(base_pair + ci * C_PAIRS, C_PAIRS)
            irow0 = pl.multiple_of(pair0 * 16 // 128, RPC // 128)
            pltpu.sync_copy(idx_hbm.at[pl.ds(irow0, NSTREAM)], idx_v)

        def fire(idx_v, rows_v, sem):
            for j in range(NSTREAM):
                pltpu.async_copy(table_hbm.at[idx_v.at[j]],
                                 rows_v.at[pl.ds(j * 128, 128)], sem)

        def drain(idx_v, rows_v, sem):
            for j in range(NSTREAM):
                pltpu.make_async_copy(table_hbm.at[idx_v.at[j]],
                                      rows_v.at[pl.ds(j * 128, 128)], sem).wait()

        def reduce_store(ci, rows_v, out_v):
            def pair_body(p, c2):
                r0 = p * 16
                v = [rows_v[r0 + t, 0:H] for t in range(16)]
                s1 = [v[2 * t] + v[2 * t + 1] for t in range(8)]
                s2 = [s1[2 * t] + s1[2 * t + 1] for t in range(4)]
                s3 = [s2[0] + s2[1], s2[2] + s2[3]]
                out_v[p, 0:H] = s3[0] + s3[1]
                return c2

            lax.fori_loop(0, C_PAIRS, pair_body, 0)
            pair0 = pl.multiple_of(base_pair + ci * C_PAIRS, C_PAIRS)
            pltpu.sync_copy(out_v, out_hbm.at[pl.ds(pair0, C_PAIRS)])

        load_idx(0, idx_v0)
        fire(idx_v0, rows_v0, sem0)

        def body2(i, carry):
            c0 = i * 2
            load_idx(c0 + 1, idx_v1)
            fire(idx_v1, rows_v1, sem1)
            drain(idx_v0, rows_v0, sem0)
            reduce_store(c0, rows_v0, out_v0)

            @pl.when(i < nhalf - 1)
            def _():
                load_idx(c0 + 2, idx_v0)
                fire(idx_v0, rows_v0, sem0)

            drain(idx_v1, rows_v1, sem1)
            reduce_store(c0 + 1, rows_v1, out_v1)
            return carry

        lax.fori_loop(0, nhalf, body2, 0)

    return k(table, idx2)


def _asm_body(ab_ref, int_ref, tok_ref, out_ref):
    x = int_ref[0].astype(jnp.float32)
    ii = lax.broadcasted_iota(jnp.int32, (H, H), 0)
    jj = lax.broadcasted_iota(jnp.int32, (H, H), 1)
    eye = (ii == jj).astype(jnp.float32)
    t = lax.dot_general(eye, x, (((1,), (1,)), ((), ())),
                        preferred_element_type=jnp.float32)
    n = ab_ref.shape[1] - 1
    t = t.reshape(H, n, n)
    ab = ab_ref[0]
    tok = tok_ref[0, :]
    interior = t + 2.0 * ab[1:, 1:][None, :, :]
    col0 = 2.0 * ab[1:, 0][None, :] + tok[:, None]
    row0 = 2.0 * ab[0, :][None, :] + tok[:, None]
    body = jnp.concatenate([col0[:, :, None], interior], axis=2)
    out = jnp.concatenate([row0[:, None, :], body], axis=1)
    out_ref[0] = out


def _assemble(attn_bias, interior3, gtw):
    b, np1, _ = attn_bias.shape
    n = np1 - 1
    return pl.pallas_call(
        _asm_body,
        grid=(b,),
        in_specs=[
            pl.BlockSpec((1, np1, np1), lambda i: (i, 0, 0)),
            pl.BlockSpec((1, n * n, H), lambda i: (i, 0, 0)),
            pl.BlockSpec((1, H), lambda i: (0, 0)),
        ],
        out_specs=pl.BlockSpec((1, H, np1, np1), lambda i: (i, 0, 0, 0)),
        out_shape=jax.ShapeDtypeStruct((b, H, np1, np1), jnp.float32),
    )(attn_bias, interior3, gtw)


def kernel(attn_bias, spatial_pos, x, edge_input, attn_edge_type,
           edge_encoder_w, edge_dis_encoder_w, spatial_pos_encoder_w,
           graph_token_w):
    b, np1, _ = attn_bias.shape
    n = np1 - 1
    p_total = b * n * n

    dis5 = edge_dis_encoder_w.reshape(-1, H, H)[:D_MAX]
    table = _build_table(edge_encoder_w, dis5, spatial_pos_encoder_w)

    sp_r = spatial_pos.astype(jnp.int32).reshape(p_total, 1)
    edge_r = edge_input.astype(jnp.int32)[..., :D_MAX, :].reshape(
        p_total, D_MAX * F)
    idx = _build_idx(sp_r, edge_r, p_total)
    idx2 = idx.reshape(p_total * 16 // 128, 128)

    interior = _sc_gather_sum(table, idx2, p_total)
    return _assemble(attn_bias, interior.reshape(b, n * n, H), graph_token_w)
